# baseline (speedup 1.0000x reference)
import jax
import jax.numpy as jnp
from jax import lax
from jax.experimental import pallas as pl
from jax.experimental.pallas import tpu as pltpu
from jax.experimental.pallas import tpu_sc as plsc

N = 100000
E = 3200000
DE = 16
G = 8
HID = 256
NODE_OUT = 128

NTILES = 32
PERP = 3136
NP = NTILES * PERP
TRASH = PERP
ACCR = PERP + 16
CHUNK = 1600
NCH = E // CHUNK
VSTEPS = CHUNK // 16
GPW = 8
WROWS = GPW * 16
NQ = ACCR // 16
EIDP = CHUNK + WROWS
FB = 112


def _seg_body(col_hbm, attr_hbm, sum_out, max_out, mean_out,
              sum_ref, max_ref, cnt_ref,
              cols_a, cols_b, cols_c, pk_a, pk_b, pk_c,
              idx_a, idx_b, idx_c, rows_a, rows_b, rows_c,
              csem_a, csem_b, csem_c, gsem_a, gsem_b, gsem_c):
    wid = lax.axis_index("s") * 2 + lax.axis_index("c")
    lo = wid * PERP

    zf = jnp.zeros((16,), jnp.float32)
    ninf = jnp.full((16,), -jnp.inf, jnp.float32)
    zi = jnp.zeros((16,), jnp.int32)
    trash_pk = jnp.full((16,), TRASH * 4096, jnp.int32)
    iota = lax.broadcasted_iota(jnp.int32, (16,), 0)

    def init_acc(i, carry):
        sum_ref[i] = zf
        max_ref[i] = ninf
        return carry
    lax.fori_loop(0, ACCR, init_acc, 0)

    def init_cnt(i, carry):
        cnt_ref[i] = zf
        return carry
    lax.fori_loop(0, NQ, init_cnt, 0)

    def init_eid(i, carry):
        pk_a[pl.ds(i * 16, 16)] = trash_pk
        pk_b[pl.ds(i * 16, 16)] = trash_pk
        pk_c[pl.ds(i * 16, 16)] = trash_pk
        return carry
    lax.fori_loop(0, EIDP // 16, init_eid, 0)

    def fetch_cols(buf, sem, cidx):
        cc = jnp.minimum(cidx, NCH - 1)
        pltpu.async_copy(col_hbm.at[pl.ds(cc * CHUNK, CHUNK)], buf, sem)

    def wait_cols(buf, sem):
        pltpu.make_async_copy(col_hbm.at[pl.ds(0, CHUNK)], buf, sem).wait()

    def compact(cols_ref, pk_ref, c):
        def step(v, cursor):
            rel = v * 16
            cv = cols_ref[pl.ds(rel, 16)]
            d = cv - lo
            m = plsc.bitcast(d, jnp.uint32) < jnp.uint32(PERP)
            k = plsc.all_reduce_population_count(m)[0]
            packed = (d * 4096) + rel + iota
            plsc.store_compressed(pk_ref.at[pl.ds(cursor, 16)], packed,
                                  mask=m)
            return cursor + k
        m = lax.fori_loop(0, VSTEPS, step, jnp.int32(0))
        pk_ref[pl.ds(m, 16)] = trash_pk
        return m

    def fire_groups(pk_ref, idx_ref, rows_ref, sem, base_g, ng, cbase):
        def fg(t, carry):
            pk = pk_ref[pl.ds((base_g + t) * 16, 16)]
            idx_ref[t] = (pk & 4095) + cbase
            pltpu.async_copy(attr_hbm.at[idx_ref.at[t]],
                             rows_ref.at[pl.ds(t * 16, 16)], sem)
            return carry
        lax.fori_loop(0, ng, fg, 0)

    def drain_groups(idx_ref, rows_ref, sem, ng):
        def dg(t, carry):
            pltpu.make_async_copy(attr_hbm.at[idx_ref.at[t]],
                                  rows_ref.at[pl.ds(t * 16, 16)], sem).wait()
            return carry
        lax.fori_loop(0, ng, dg, 0)

    def accumulate(pk_ref, rows_ref, base_g, ng):
        def g_body(g, carry):
            nv = pk_ref[pl.ds((base_g + g) * 16, 16)] // 4096
            for j in range(16):
                n = nv[j]
                row = rows_ref[g * 16 + j]
                sum_ref[n] = sum_ref[n] + row
                max_ref[n] = jnp.maximum(max_ref[n], row)
                q = n // 16
                lane = n - q * 16
                cnt_ref[q] = cnt_ref[q] + (iota == lane).astype(jnp.float32)
            return carry
        lax.fori_loop(0, ng, g_body, 0)

    def ngroups(m):
        return (m + 15) // 16

    def finish_chunk(pk_ref, idx_ref, rows_ref, sem, m, cbase):
        ngt = ngroups(m)
        n0 = jnp.minimum(ngt, GPW)
        drain_groups(idx_ref, rows_ref, sem, n0)
        accumulate(pk_ref, rows_ref, 0, n0)
        nw = (ngt + GPW - 1) // GPW

        def wave(w, carry):
            base = w * GPW
            nn = jnp.minimum(GPW, ngt - base)
            fire_groups(pk_ref, idx_ref, rows_ref, sem, base, nn, cbase)
            drain_groups(idx_ref, rows_ref, sem, nn)
            accumulate(pk_ref, rows_ref, base, nn)
            return carry
        lax.fori_loop(1, nw, wave, 0)

    def half(c, colsX, csemX, pkX, idxX, rowsX, gsemX,
             pkZ, idxZ, rowsZ, gsemZ, m_prev2):
        wait_cols(colsX, csemX)
        m = compact(colsX, pkX, c)
        fire_groups(pkX, idxX, rowsX, gsemX, 0,
                    jnp.minimum(ngroups(m), GPW), c * CHUNK)
        fetch_cols(colsX, csemX, c + 3)
        finish_chunk(pkZ, idxZ, rowsZ, gsemZ, m_prev2, (c - 2) * CHUNK)
        return m

    fetch_cols(cols_a, csem_a, 0)
    fetch_cols(cols_b, csem_b, 1)
    fetch_cols(cols_c, csem_c, 2)
    wait_cols(cols_a, csem_a)
    m0 = compact(cols_a, pk_a, 0)
    fire_groups(pk_a, idx_a, rows_a, gsem_a, 0,
                jnp.minimum(ngroups(m0), GPW), 0)
    fetch_cols(cols_a, csem_a, 3)
    wait_cols(cols_b, csem_b)
    m1 = compact(cols_b, pk_b, 1)
    fire_groups(pk_b, idx_b, rows_b, gsem_b, 0,
                jnp.minimum(ngroups(m1), GPW), CHUNK)
    fetch_cols(cols_b, csem_b, 4)

    def triple(p, carry):
        mA, mB = carry
        mC = half(3 * p + 2, cols_c, csem_c, pk_c, idx_c, rows_c,
                  gsem_c, pk_a, idx_a, rows_a, gsem_a, mA)
        mA2 = half(3 * p + 3, cols_a, csem_a, pk_a, idx_a, rows_a,
                   gsem_a, pk_b, idx_b, rows_b, gsem_b, mB)
        mB2 = half(3 * p + 4, cols_b, csem_b, pk_b, idx_b, rows_b,
                   gsem_b, pk_c, idx_c, rows_c, gsem_c, mC)
        return (mA2, mB2)
    mA, mB = lax.fori_loop(0, (NCH - 2) // 3, triple, (m0, m1))

    finish_chunk(pk_a, idx_a, rows_a, gsem_a, mA, (NCH - 2) * CHUNK)
    finish_chunk(pk_b, idx_b, rows_b, gsem_b, mB, (NCH - 1) * CHUNK)
    wait_cols(cols_a, csem_a)
    wait_cols(cols_b, csem_b)
    wait_cols(cols_c, csem_c)

    def fin_blk(b, carry):
        for qq in range(FB // 16):
            cv = cnt_ref[b * (FB // 16) + qq]
            for j in range(16):
                g = (b * (FB // 16) + qq) * 16 + j
                cb = lax.broadcast_in_dim(cv[j], (16,), ())
                rows_a[qq * 16 + j] = sum_ref[g] / jnp.maximum(cb, 1.0)
                max_ref[g] = jnp.where(cb > 0.0, max_ref[g], zf)
        pltpu.sync_copy(rows_a.at[pl.ds(0, FB)],
                        mean_out.at[pl.ds(lo + b * FB, FB)])
        return carry
    lax.fori_loop(0, PERP // FB, fin_blk, 0)

    pltpu.sync_copy(sum_ref.at[pl.ds(0, PERP)], sum_out.at[pl.ds(lo, PERP)])
    pltpu.sync_copy(max_ref.at[pl.ds(0, PERP)], max_out.at[pl.ds(lo, PERP)])


@jax.jit
def _segment_reduce(col, attr):
    f = pl.kernel(
        _seg_body,
        out_type=[
            jax.ShapeDtypeStruct((NP, DE), jnp.float32),
            jax.ShapeDtypeStruct((NP, DE), jnp.float32),
            jax.ShapeDtypeStruct((NP, DE), jnp.float32),
        ],
        mesh=plsc.VectorSubcoreMesh(core_axis_name="c", subcore_axis_name="s"),
        compiler_params=pltpu.CompilerParams(use_tc_tiling_on_sc=False,
                                             needs_layout_passes=False),
        scratch_types=[
            pltpu.VMEM((ACCR, DE), jnp.float32),
            pltpu.VMEM((ACCR, DE), jnp.float32),
            pltpu.VMEM((NQ, 16), jnp.float32),
            pltpu.VMEM((CHUNK,), jnp.int32),
            pltpu.VMEM((CHUNK,), jnp.int32),
            pltpu.VMEM((CHUNK,), jnp.int32),
            pltpu.VMEM((EIDP,), jnp.int32),
            pltpu.VMEM((EIDP,), jnp.int32),
            pltpu.VMEM((EIDP,), jnp.int32),
            pltpu.VMEM((GPW, 16), jnp.int32),
            pltpu.VMEM((GPW, 16), jnp.int32),
            pltpu.VMEM((GPW, 16), jnp.int32),
            pltpu.VMEM((WROWS, DE), jnp.float32),
            pltpu.VMEM((WROWS, DE), jnp.float32),
            pltpu.VMEM((WROWS, DE), jnp.float32),
            pltpu.SemaphoreType.DMA,
            pltpu.SemaphoreType.DMA,
            pltpu.SemaphoreType.DMA,
            pltpu.SemaphoreType.DMA,
            pltpu.SemaphoreType.DMA,
            pltpu.SemaphoreType.DMA,
        ],
    )
    return f(col, attr)


BN = 1000


def _mlp_body(s_ref, mx_ref, mn_ref, b_ref, u_ref, w1_ref, b1_ref, w2_ref,
              b2_ref, o_ref):
    h = jnp.dot(s_ref[...], w1_ref[0:DE, :],
                preferred_element_type=jnp.float32)
    h = h + jnp.dot(mx_ref[...], w1_ref[DE:2 * DE, :],
                    preferred_element_type=jnp.float32)
    h = h + jnp.dot(mn_ref[...], w1_ref[2 * DE:3 * DE, :],
                    preferred_element_type=jnp.float32)
    bv = b_ref[0, 0, :]
    ub = jnp.zeros((BN,), jnp.float32)
    for g in range(G):
        ub = ub + jnp.where(bv == g, u_ref[g, 0], 0.0)
    h = h + ub[:, None] * w1_ref[3 * DE:3 * DE + 1, :]
    h = jnp.maximum(h + b1_ref[...], 0.0)
    o_ref[...] = jnp.dot(h, w2_ref[...],
                         preferred_element_type=jnp.float32) + b2_ref[...]


@jax.jit
def _mlp(sums, maxs, means, batch, u, W1, b1, W2, b2):
    batch3 = batch.reshape(N // BN, 1, BN)
    return pl.pallas_call(
        _mlp_body,
        grid=(N // BN,),
        in_specs=[
            pl.BlockSpec((BN, DE), lambda i: (i, 0)),
            pl.BlockSpec((BN, DE), lambda i: (i, 0)),
            pl.BlockSpec((BN, DE), lambda i: (i, 0)),
            pl.BlockSpec((1, 1, BN), lambda i: (i, 0, 0)),
            pl.BlockSpec((G, 1), lambda i: (0, 0)),
            pl.BlockSpec((3 * DE + 1, HID), lambda i: (0, 0)),
            pl.BlockSpec((1, HID), lambda i: (0, 0)),
            pl.BlockSpec((HID, NODE_OUT), lambda i: (0, 0)),
            pl.BlockSpec((1, NODE_OUT), lambda i: (0, 0)),
        ],
        out_specs=pl.BlockSpec((BN, NODE_OUT), lambda i: (i, 0)),
        out_shape=jax.ShapeDtypeStruct((N, NODE_OUT), jnp.float32),
    )(sums, maxs, means, batch3, u, W1, b1.reshape(1, HID), W2,
      b2.reshape(1, NODE_OUT))


def kernel(x, edge_index, edge_attr, u, batch, W1, b1, W2, b2):
    col = edge_index[1]
    sums, maxs, means = _segment_reduce(col, edge_attr)
    return _mlp(sums, maxs, means, batch, u, W1, b1, W2, b2)


# speedup vs baseline: 1.0996x; 1.0996x over previous
import jax
import jax.numpy as jnp
from jax import lax
from jax.experimental import pallas as pl
from jax.experimental.pallas import tpu as pltpu
from jax.experimental.pallas import tpu_sc as plsc

N = 100000
E = 3200000
DE = 16
G = 8
HID = 256
NODE_OUT = 128

NTILES = 32
PERP = 3136
NP = NTILES * PERP
TRASH = PERP
ACCR = PERP + 16
CHUNK = 4000
NCH = E // CHUNK
VSTEPS = CHUNK // 16
GPW = 10
WROWS = GPW * 16
NQ = ACCR // 16
EIDP = CHUNK + WROWS
FB = 112


def _seg_body(col_hbm, attr_hbm, sum_out, max_out, mean_out,
              sum_ref, max_ref, cnt_ref,
              cols_a, cols_b, pk_a, pk_b,
              idx_a, idx_b, rows_a, rows_b,
              csem_a, csem_b, gsem_a, gsem_b):
    wid = lax.axis_index("s") * 2 + lax.axis_index("c")
    lo = wid * PERP

    zf = jnp.zeros((16,), jnp.float32)
    ninf = jnp.full((16,), -jnp.inf, jnp.float32)
    zi = jnp.zeros((16,), jnp.int32)
    trash_pk = jnp.full((16,), TRASH * 4096, jnp.int32)
    iota = lax.broadcasted_iota(jnp.int32, (16,), 0)

    def init_acc(i, carry):
        sum_ref[i] = zf
        max_ref[i] = ninf
        return carry
    lax.fori_loop(0, ACCR, init_acc, 0)

    def init_cnt(i, carry):
        cnt_ref[i] = zf
        return carry
    lax.fori_loop(0, NQ, init_cnt, 0)

    def init_eid(i, carry):
        pk_a[pl.ds(i * 16, 16)] = trash_pk
        pk_b[pl.ds(i * 16, 16)] = trash_pk
        return carry
    lax.fori_loop(0, EIDP // 16, init_eid, 0)

    def fetch_cols(buf, sem, cidx):
        cc = jnp.minimum(cidx, NCH - 1)
        pltpu.async_copy(col_hbm.at[pl.ds(cc * CHUNK, CHUNK)], buf, sem)

    def wait_cols(buf, sem):
        pltpu.make_async_copy(col_hbm.at[pl.ds(0, CHUNK)], buf, sem).wait()

    def compact(cols_ref, pk_ref, c):
        def step(v, cursor):
            rel = v * 16
            cv = cols_ref[pl.ds(rel, 16)]
            d = cv - lo
            m = plsc.bitcast(d, jnp.uint32) < jnp.uint32(PERP)
            k = plsc.all_reduce_population_count(m)[0]
            packed = (d * 4096) + rel + iota
            plsc.store_compressed(pk_ref.at[pl.ds(cursor, 16)], packed,
                                  mask=m)
            return cursor + k
        m = lax.fori_loop(0, VSTEPS, step, jnp.int32(0))
        pk_ref[pl.ds(m, 16)] = trash_pk
        return m

    def fire_groups(pk_ref, idx_ref, rows_ref, sem, base_g, ng, cbase):
        def fg(t, carry):
            pk = pk_ref[pl.ds((base_g + t) * 16, 16)]
            idx_ref[t] = (pk & 4095) + cbase
            pltpu.async_copy(attr_hbm.at[idx_ref.at[t]],
                             rows_ref.at[pl.ds(t * 16, 16)], sem)
            return carry
        lax.fori_loop(0, ng, fg, 0)

    def drain_groups(idx_ref, rows_ref, sem, ng):
        def dg(t, carry):
            pltpu.make_async_copy(attr_hbm.at[idx_ref.at[t]],
                                  rows_ref.at[pl.ds(t * 16, 16)], sem).wait()
            return carry
        lax.fori_loop(0, ng, dg, 0)

    def accumulate(pk_ref, rows_ref, base_g, ng):
        def g_body(g, carry):
            nv = pk_ref[pl.ds((base_g + g) * 16, 16)] // 4096
            for j in range(16):
                n = nv[j]
                row = rows_ref[g * 16 + j]
                sum_ref[n] = sum_ref[n] + row
                max_ref[n] = jnp.maximum(max_ref[n], row)
                q = n // 16
                lane = n - q * 16
                cnt_ref[q] = cnt_ref[q] + (iota == lane).astype(jnp.float32)
            return carry
        lax.fori_loop(0, ng, g_body, 0)

    def ngroups(m):
        return (m + 15) // 16

    def finish_chunk(pk_ref, idx_ref, rows_ref, sem, m, cbase):
        ngt = ngroups(m)
        n0 = jnp.minimum(ngt, GPW)
        drain_groups(idx_ref, rows_ref, sem, n0)
        accumulate(pk_ref, rows_ref, 0, n0)
        nw = (ngt + GPW - 1) // GPW

        def wave(w, carry):
            base = w * GPW
            nn = jnp.minimum(GPW, ngt - base)
            fire_groups(pk_ref, idx_ref, rows_ref, sem, base, nn, cbase)
            drain_groups(idx_ref, rows_ref, sem, nn)
            accumulate(pk_ref, rows_ref, base, nn)
            return carry
        lax.fori_loop(1, nw, wave, 0)

    def half(c, colsX, csemX, pkX, idxX, rowsX, gsemX,
             pkY, idxY, rowsY, gsemY, m_prev):
        wait_cols(colsX, csemX)
        m = compact(colsX, pkX, c)
        fire_groups(pkX, idxX, rowsX, gsemX, 0,
                    jnp.minimum(ngroups(m), GPW), c * CHUNK)
        fetch_cols(colsX, csemX, c + 2)
        finish_chunk(pkY, idxY, rowsY, gsemY, m_prev, (c - 1) * CHUNK)
        return m

    fetch_cols(cols_a, csem_a, 0)
    fetch_cols(cols_b, csem_b, 1)
    wait_cols(cols_a, csem_a)
    m_prev = compact(cols_a, pk_a, 0)
    fire_groups(pk_a, idx_a, rows_a, gsem_a, 0,
                jnp.minimum(ngroups(m_prev), GPW), 0)
    fetch_cols(cols_a, csem_a, 2)

    def pair(p, mp):
        m1 = half(2 * p + 1, cols_b, csem_b, pk_b, idx_b, rows_b,
                  gsem_b, pk_a, idx_a, rows_a, gsem_a, mp)
        m2 = half(2 * p + 2, cols_a, csem_a, pk_a, idx_a, rows_a,
                  gsem_a, pk_b, idx_b, rows_b, gsem_b, m1)
        return m2
    m_prev = lax.fori_loop(0, (NCH - 2) // 2, pair, m_prev)

    m_last = half(NCH - 1, cols_b, csem_b, pk_b, idx_b, rows_b,
                  gsem_b, pk_a, idx_a, rows_a, gsem_a, m_prev)
    finish_chunk(pk_b, idx_b, rows_b, gsem_b, m_last, (NCH - 1) * CHUNK)
    wait_cols(cols_a, csem_a)
    wait_cols(cols_b, csem_b)

    def fin_blk(b, carry):
        for qq in range(FB // 16):
            cv = cnt_ref[b * (FB // 16) + qq]
            for j in range(16):
                g = (b * (FB // 16) + qq) * 16 + j
                cb = lax.broadcast_in_dim(cv[j], (16,), ())
                rows_a[qq * 16 + j] = sum_ref[g] / jnp.maximum(cb, 1.0)
                max_ref[g] = jnp.where(cb > 0.0, max_ref[g], zf)
        pltpu.sync_copy(rows_a.at[pl.ds(0, FB)],
                        mean_out.at[pl.ds(lo + b * FB, FB)])
        return carry
    lax.fori_loop(0, PERP // FB, fin_blk, 0)

    pltpu.sync_copy(sum_ref.at[pl.ds(0, PERP)], sum_out.at[pl.ds(lo, PERP)])
    pltpu.sync_copy(max_ref.at[pl.ds(0, PERP)], max_out.at[pl.ds(lo, PERP)])


@jax.jit
def _segment_reduce(col, attr):
    f = pl.kernel(
        _seg_body,
        out_type=[
            jax.ShapeDtypeStruct((NP, DE), jnp.float32),
            jax.ShapeDtypeStruct((NP, DE), jnp.float32),
            jax.ShapeDtypeStruct((NP, DE), jnp.float32),
        ],
        mesh=plsc.VectorSubcoreMesh(core_axis_name="c", subcore_axis_name="s"),
        compiler_params=pltpu.CompilerParams(use_tc_tiling_on_sc=False,
                                             needs_layout_passes=False),
        scratch_types=[
            pltpu.VMEM((ACCR, DE), jnp.float32),
            pltpu.VMEM((ACCR, DE), jnp.float32),
            pltpu.VMEM((NQ, 16), jnp.float32),
            pltpu.VMEM((CHUNK,), jnp.int32),
            pltpu.VMEM((CHUNK,), jnp.int32),
            pltpu.VMEM((EIDP,), jnp.int32),
            pltpu.VMEM((EIDP,), jnp.int32),
            pltpu.VMEM((GPW, 16), jnp.int32),
            pltpu.VMEM((GPW, 16), jnp.int32),
            pltpu.VMEM((WROWS, DE), jnp.float32),
            pltpu.VMEM((WROWS, DE), jnp.float32),
            pltpu.SemaphoreType.DMA,
            pltpu.SemaphoreType.DMA,
            pltpu.SemaphoreType.DMA,
            pltpu.SemaphoreType.DMA,
        ],
    )
    return f(col, attr)


BN = 1000


def _mlp_body(s_ref, mx_ref, mn_ref, b_ref, u_ref, w1_ref, b1_ref, w2_ref,
              b2_ref, o_ref):
    h = jnp.dot(s_ref[...], w1_ref[0:DE, :],
                preferred_element_type=jnp.float32)
    h = h + jnp.dot(mx_ref[...], w1_ref[DE:2 * DE, :],
                    preferred_element_type=jnp.float32)
    h = h + jnp.dot(mn_ref[...], w1_ref[2 * DE:3 * DE, :],
                    preferred_element_type=jnp.float32)
    bv = b_ref[0, 0, :]
    ub = jnp.zeros((BN,), jnp.float32)
    for g in range(G):
        ub = ub + jnp.where(bv == g, u_ref[g, 0], 0.0)
    h = h + ub[:, None] * w1_ref[3 * DE:3 * DE + 1, :]
    h = jnp.maximum(h + b1_ref[...], 0.0)
    o_ref[...] = jnp.dot(h, w2_ref[...],
                         preferred_element_type=jnp.float32) + b2_ref[...]


@jax.jit
def _mlp(sums, maxs, means, batch, u, W1, b1, W2, b2):
    batch3 = batch.reshape(N // BN, 1, BN)
    return pl.pallas_call(
        _mlp_body,
        grid=(N // BN,),
        in_specs=[
            pl.BlockSpec((BN, DE), lambda i: (i, 0)),
            pl.BlockSpec((BN, DE), lambda i: (i, 0)),
            pl.BlockSpec((BN, DE), lambda i: (i, 0)),
            pl.BlockSpec((1, 1, BN), lambda i: (i, 0, 0)),
            pl.BlockSpec((G, 1), lambda i: (0, 0)),
            pl.BlockSpec((3 * DE + 1, HID), lambda i: (0, 0)),
            pl.BlockSpec((1, HID), lambda i: (0, 0)),
            pl.BlockSpec((HID, NODE_OUT), lambda i: (0, 0)),
            pl.BlockSpec((1, NODE_OUT), lambda i: (0, 0)),
        ],
        out_specs=pl.BlockSpec((BN, NODE_OUT), lambda i: (i, 0)),
        out_shape=jax.ShapeDtypeStruct((N, NODE_OUT), jnp.float32),
    )(sums, maxs, means, batch3, u, W1, b1.reshape(1, HID), W2,
      b2.reshape(1, NODE_OUT))


def kernel(x, edge_index, edge_attr, u, batch, W1, b1, W2, b2):
    col = edge_index[1]
    sums, maxs, means = _segment_reduce(col, edge_attr)
    return _mlp(sums, maxs, means, batch, u, W1, b1, W2, b2)


# speedup vs baseline: 1.0999x; 1.0003x over previous
import jax
import jax.numpy as jnp
from jax import lax
from jax.experimental import pallas as pl
from jax.experimental.pallas import tpu as pltpu
from jax.experimental.pallas import tpu_sc as plsc

N = 100000
E = 3200000
DE = 16
G = 8
HID = 256
NODE_OUT = 128

NTILES = 32
PERP = 3136
NP = NTILES * PERP
TRASH = PERP
ACCR = PERP + 16
CHUNK = 4000
NCH = E // CHUNK
VSTEPS = CHUNK // 16
GPW = 10
WROWS = GPW * 16
NQ = ACCR // 16
EIDP = CHUNK + WROWS
FB = 112


def _seg_body(col_hbm, attr_hbm, out3,
              sum_ref, max_ref, cnt_ref,
              cols_a, cols_b, pk_a, pk_b,
              idx_a, idx_b, rows_a, rows_b,
              csem_a, csem_b, gsem_a, gsem_b):
    wid = lax.axis_index("s") * 2 + lax.axis_index("c")
    lo = wid * PERP

    zf = jnp.zeros((16,), jnp.float32)
    ninf = jnp.full((16,), -jnp.inf, jnp.float32)
    zi = jnp.zeros((16,), jnp.int32)
    trash_pk = jnp.full((16,), TRASH * 4096, jnp.int32)
    iota = lax.broadcasted_iota(jnp.int32, (16,), 0)

    def init_acc(i, carry):
        sum_ref[i] = zf
        max_ref[i] = ninf
        return carry
    lax.fori_loop(0, ACCR, init_acc, 0)

    def init_cnt(i, carry):
        cnt_ref[i] = zf
        return carry
    lax.fori_loop(0, NQ, init_cnt, 0)

    def init_eid(i, carry):
        pk_a[pl.ds(i * 16, 16)] = trash_pk
        pk_b[pl.ds(i * 16, 16)] = trash_pk
        return carry
    lax.fori_loop(0, EIDP // 16, init_eid, 0)

    def fetch_cols(buf, sem, cidx):
        cc = jnp.minimum(cidx, NCH - 1)
        pltpu.async_copy(col_hbm.at[pl.ds(cc * CHUNK, CHUNK)], buf, sem)

    def wait_cols(buf, sem):
        pltpu.make_async_copy(col_hbm.at[pl.ds(0, CHUNK)], buf, sem).wait()

    def compact(cols_ref, pk_ref, c):
        def step(v, cursor):
            rel = v * 16
            cv = cols_ref[pl.ds(rel, 16)]
            d = cv - lo
            m = plsc.bitcast(d, jnp.uint32) < jnp.uint32(PERP)
            k = plsc.all_reduce_population_count(m)[0]
            packed = (d * 4096) + rel + iota
            plsc.store_compressed(pk_ref.at[pl.ds(cursor, 16)], packed,
                                  mask=m)
            return cursor + k
        m = lax.fori_loop(0, VSTEPS, step, jnp.int32(0))
        pk_ref[pl.ds(m, 16)] = trash_pk
        return m

    def fire_groups(pk_ref, idx_ref, rows_ref, sem, base_g, ng, cbase):
        def fg(t, carry):
            pk = pk_ref[pl.ds((base_g + t) * 16, 16)]
            idx_ref[t] = (pk & 4095) + cbase
            pltpu.async_copy(attr_hbm.at[idx_ref.at[t]],
                             rows_ref.at[pl.ds(t * 16, 16)], sem)
            return carry
        lax.fori_loop(0, ng, fg, 0)

    def drain_groups(idx_ref, rows_ref, sem, ng):
        def dg(t, carry):
            pltpu.make_async_copy(attr_hbm.at[idx_ref.at[t]],
                                  rows_ref.at[pl.ds(t * 16, 16)], sem).wait()
            return carry
        lax.fori_loop(0, ng, dg, 0)

    def accumulate(pk_ref, rows_ref, base_g, ng):
        def g_body(g, carry):
            nv = pk_ref[pl.ds((base_g + g) * 16, 16)] // 4096
            for j in range(16):
                n = nv[j]
                row = rows_ref[g * 16 + j]
                sum_ref[n] = sum_ref[n] + row
                max_ref[n] = jnp.maximum(max_ref[n], row)
                q = n // 16
                lane = n - q * 16
                cnt_ref[q] = cnt_ref[q] + (iota == lane).astype(jnp.float32)
            return carry
        lax.fori_loop(0, ng, g_body, 0)

    def ngroups(m):
        return (m + 15) // 16

    def finish_chunk(pk_ref, idx_ref, rows_ref, sem, m, cbase):
        ngt = ngroups(m)
        n0 = jnp.minimum(ngt, GPW)
        drain_groups(idx_ref, rows_ref, sem, n0)
        accumulate(pk_ref, rows_ref, 0, n0)
        nw = (ngt + GPW - 1) // GPW

        def wave(w, carry):
            base = w * GPW
            nn = jnp.minimum(GPW, ngt - base)
            fire_groups(pk_ref, idx_ref, rows_ref, sem, base, nn, cbase)
            drain_groups(idx_ref, rows_ref, sem, nn)
            accumulate(pk_ref, rows_ref, base, nn)
            return carry
        lax.fori_loop(1, nw, wave, 0)

    def half(c, colsX, csemX, pkX, idxX, rowsX, gsemX,
             pkY, idxY, rowsY, gsemY, m_prev):
        wait_cols(colsX, csemX)
        m = compact(colsX, pkX, c)
        fire_groups(pkX, idxX, rowsX, gsemX, 0,
                    jnp.minimum(ngroups(m), GPW), c * CHUNK)
        fetch_cols(colsX, csemX, c + 2)
        finish_chunk(pkY, idxY, rowsY, gsemY, m_prev, (c - 1) * CHUNK)
        return m

    fetch_cols(cols_a, csem_a, 0)
    fetch_cols(cols_b, csem_b, 1)
    wait_cols(cols_a, csem_a)
    m_prev = compact(cols_a, pk_a, 0)
    fire_groups(pk_a, idx_a, rows_a, gsem_a, 0,
                jnp.minimum(ngroups(m_prev), GPW), 0)
    fetch_cols(cols_a, csem_a, 2)

    def pair(p, mp):
        m1 = half(2 * p + 1, cols_b, csem_b, pk_b, idx_b, rows_b,
                  gsem_b, pk_a, idx_a, rows_a, gsem_a, mp)
        m2 = half(2 * p + 2, cols_a, csem_a, pk_a, idx_a, rows_a,
                  gsem_a, pk_b, idx_b, rows_b, gsem_b, m1)
        return m2
    m_prev = lax.fori_loop(0, (NCH - 2) // 2, pair, m_prev)

    m_last = half(NCH - 1, cols_b, csem_b, pk_b, idx_b, rows_b,
                  gsem_b, pk_a, idx_a, rows_a, gsem_a, m_prev)
    finish_chunk(pk_b, idx_b, rows_b, gsem_b, m_last, (NCH - 1) * CHUNK)
    wait_cols(cols_a, csem_a)
    wait_cols(cols_b, csem_b)

    def fin_blk(b, carry):
        for qq in range(FB // 16):
            cv = cnt_ref[b * (FB // 16) + qq]
            for j in range(16):
                g = (b * (FB // 16) + qq) * 16 + j
                cb = lax.broadcast_in_dim(cv[j], (16,), ())
                rows_a[qq * 16 + j] = sum_ref[g] / jnp.maximum(cb, 1.0)
                max_ref[g] = jnp.where(cb > 0.0, max_ref[g], zf)
        pltpu.sync_copy(rows_a.at[pl.ds(0, FB)],
                        out3.at[2, pl.ds(lo + b * FB, FB)])
        return carry
    lax.fori_loop(0, PERP // FB, fin_blk, 0)

    pltpu.sync_copy(sum_ref.at[pl.ds(0, PERP)],
                    out3.at[0, pl.ds(lo, PERP)])
    pltpu.sync_copy(max_ref.at[pl.ds(0, PERP)],
                    out3.at[1, pl.ds(lo, PERP)])


@jax.jit
def _segment_reduce(col, attr):
    f = pl.kernel(
        _seg_body,
        out_type=[
            jax.ShapeDtypeStruct((3, NP, DE), jnp.float32),
        ],
        mesh=plsc.VectorSubcoreMesh(core_axis_name="c", subcore_axis_name="s"),
        compiler_params=pltpu.CompilerParams(use_tc_tiling_on_sc=False,
                                             needs_layout_passes=False),
        scratch_types=[
            pltpu.VMEM((ACCR, DE), jnp.float32),
            pltpu.VMEM((ACCR, DE), jnp.float32),
            pltpu.VMEM((NQ, 16), jnp.float32),
            pltpu.VMEM((CHUNK,), jnp.int32),
            pltpu.VMEM((CHUNK,), jnp.int32),
            pltpu.VMEM((EIDP,), jnp.int32),
            pltpu.VMEM((EIDP,), jnp.int32),
            pltpu.VMEM((GPW, 16), jnp.int32),
            pltpu.VMEM((GPW, 16), jnp.int32),
            pltpu.VMEM((WROWS, DE), jnp.float32),
            pltpu.VMEM((WROWS, DE), jnp.float32),
            pltpu.SemaphoreType.DMA,
            pltpu.SemaphoreType.DMA,
            pltpu.SemaphoreType.DMA,
            pltpu.SemaphoreType.DMA,
        ],
    )
    return f(col, attr)[0]


BN = 1000


def _mlp_body(h3_ref, b_ref, u_ref, w1_ref, b1_ref, w2_ref,
              b2_ref, o_ref):
    h = jnp.dot(h3_ref[0], w1_ref[0:DE, :],
                preferred_element_type=jnp.float32)
    h = h + jnp.dot(h3_ref[1], w1_ref[DE:2 * DE, :],
                    preferred_element_type=jnp.float32)
    h = h + jnp.dot(h3_ref[2], w1_ref[2 * DE:3 * DE, :],
                    preferred_element_type=jnp.float32)
    bv = b_ref[0, 0, :]
    ub = jnp.zeros((BN,), jnp.float32)
    for g in range(G):
        ub = ub + jnp.where(bv == g, u_ref[g, 0], 0.0)
    h = h + ub[:, None] * w1_ref[3 * DE:3 * DE + 1, :]
    h = jnp.maximum(h + b1_ref[...], 0.0)
    o_ref[...] = jnp.dot(h, w2_ref[...],
                         preferred_element_type=jnp.float32) + b2_ref[...]


@jax.jit
def _mlp(h3, batch, u, W1, b1, W2, b2):
    batch3 = batch.reshape(N // BN, 1, BN)
    return pl.pallas_call(
        _mlp_body,
        grid=(N // BN,),
        in_specs=[
            pl.BlockSpec((3, BN, DE), lambda i: (0, i, 0)),
            pl.BlockSpec((1, 1, BN), lambda i: (i, 0, 0)),
            pl.BlockSpec((G, 1), lambda i: (0, 0)),
            pl.BlockSpec((3 * DE + 1, HID), lambda i: (0, 0)),
            pl.BlockSpec((1, HID), lambda i: (0, 0)),
            pl.BlockSpec((HID, NODE_OUT), lambda i: (0, 0)),
            pl.BlockSpec((1, NODE_OUT), lambda i: (0, 0)),
        ],
        out_specs=pl.BlockSpec((BN, NODE_OUT), lambda i: (i, 0)),
        out_shape=jax.ShapeDtypeStruct((N, NODE_OUT), jnp.float32),
    )(h3, batch3, u, W1, b1.reshape(1, HID), W2,
      b2.reshape(1, NODE_OUT))


def kernel(x, edge_index, edge_attr, u, batch, W1, b1, W2, b2):
    col = edge_index[1]
    h3 = _segment_reduce(col, edge_attr)
    return _mlp(h3, batch, u, W1, b1, W2, b2)
